# baseline (speedup 1.0000x reference)
import functools

import jax
import jax.numpy as jnp
from jax import lax
from jax.experimental import pallas as pl
from jax.experimental.pallas import tpu as pltpu
from jax.experimental.pallas import tpu_sc as plsc

B = 16384
N_FIELDS = 26
VOCAB = 100000
EMBED_DIM = 64
N_DENSE = 13
H1 = 1024
H2 = 512
EIN = N_FIELDS * EMBED_DIM

NW = 32
RPW = B // NW
STAGE = 32
SIDX = STAGE * N_FIELDS


def _gather_body(xv_hbm, table_hbm, out_hbm, idx_v, gsem):
    wid = lax.axis_index("c") * 16 + lax.axis_index("s")
    row0 = wid * RPW

    pltpu.sync_copy(xv_hbm.at[pl.ds(row0 * N_FIELDS, RPW * N_FIELDS)],
                    idx_v.at[pl.ds(0, RPW * N_FIELDS)])

    def row(b, _):
        pos0 = b * N_FIELDS
        rg = (row0 + b) * N_FIELDS
        v0 = idx_v[pl.ds(pos0, 16)]
        v1 = idx_v[pl.ds(pos0 + 16, 16)]
        for f in range(N_FIELDS):
            idx = v0[f] if f < 16 else v1[f - 16]
            pltpu.async_copy(
                table_hbm.at[f, pl.ds(idx, 1), :],
                out_hbm.at[pl.ds(rg + f, 1)],
                gsem)
        return 0

    lax.fori_loop(0, RPW, row, 0)

    pltpu.make_async_copy(
        out_hbm.at[pl.ds(row0 * N_FIELDS, RPW * N_FIELDS)],
        out_hbm.at[pl.ds(row0 * N_FIELDS, RPW * N_FIELDS)],
        gsem).wait()


_gather = functools.partial(
    pl.kernel,
    mesh=plsc.VectorSubcoreMesh(core_axis_name="c", subcore_axis_name="s"),
    out_type=jax.ShapeDtypeStruct((B * N_FIELDS, EMBED_DIM), jnp.float32),
    scratch_types=[
        pltpu.VMEM((RPW * N_FIELDS + 16,), jnp.int32),
        pltpu.SemaphoreType.DMA,
    ],
)(_gather_body)


BB = 512


def _mlp_body(xi_ref, emb_ref, w1d_ref, w1e_ref, b1_ref, w2_ref, b2_ref,
              w3_ref, b3_ref, o_ref):
    h1 = jnp.dot(emb_ref[...], w1e_ref[...], preferred_element_type=jnp.float32)
    h1 = h1 + jnp.dot(xi_ref[...], w1d_ref[...], preferred_element_type=jnp.float32)
    h1 = jnp.maximum(h1 + b1_ref[...], 0.0)
    h2 = jnp.dot(h1, w2_ref[...], preferred_element_type=jnp.float32)
    h2 = jnp.maximum(h2 + b2_ref[...], 0.0)
    y = jnp.dot(h2, w3_ref[...], preferred_element_type=jnp.float32) + b3_ref[...]
    o_ref[...] = jax.nn.sigmoid(y)


def kernel(xi, xv, emb_tables, W1, b1, W2, b2, W3, b3):
    xv_flat = xv.reshape(B * N_FIELDS).astype(jnp.int32)

    emb_flat = _gather(xv_flat, emb_tables).reshape(B, EIN)

    W1d = W1[:N_DENSE]
    W1e = W1[N_DENSE:]

    out = pl.pallas_call(
        _mlp_body,
        grid=(B // BB,),
        in_specs=[
            pl.BlockSpec((BB, N_DENSE), lambda i: (i, 0)),
            pl.BlockSpec((BB, EIN), lambda i: (i, 0)),
            pl.BlockSpec((N_DENSE, H1), lambda i: (0, 0)),
            pl.BlockSpec((EIN, H1), lambda i: (0, 0)),
            pl.BlockSpec((1, H1), lambda i: (0, 0)),
            pl.BlockSpec((H1, H2), lambda i: (0, 0)),
            pl.BlockSpec((1, H2), lambda i: (0, 0)),
            pl.BlockSpec((H2, 1), lambda i: (0, 0)),
            pl.BlockSpec((1, 1), lambda i: (0, 0)),
        ],
        out_specs=pl.BlockSpec((BB, 1), lambda i: (i, 0)),
        out_shape=jax.ShapeDtypeStruct((B, 1), jnp.float32),
    )(xi, emb_flat, W1d, W1e, b1.reshape(1, H1), W2, b2.reshape(1, H2),
      W3, b3.reshape(1, 1))
    return out


# speedup vs baseline: 4.1852x; 4.1852x over previous
import functools

import jax
import jax.numpy as jnp
from jax import lax
from jax.experimental import pallas as pl
from jax.experimental.pallas import tpu as pltpu
from jax.experimental.pallas import tpu_sc as plsc

B = 16384
N_FIELDS = 26
VOCAB = 100000
EMBED_DIM = 64
N_DENSE = 13
H1 = 1024
H2 = 512
EIN = N_FIELDS * EMBED_DIM

NW = 32
CHUNK = 128
NCHB = B // CHUNK
NCH = N_FIELDS * NCHB // NW
IPW = NCH * CHUNK


def _gather_body(xvt_hbm, table_hbm, out_hbm, idx_v, rows_v, gsem, wsem):
    wid = lax.axis_index("c") * 16 + lax.axis_index("s")
    g0 = wid * NCH

    pltpu.sync_copy(xvt_hbm.at[pl.ds(wid * IPW, IPW)], idx_v)

    def chunk_src(c):
        g = g0 + c
        f = g // NCHB
        b0 = (g % NCHB) * CHUNK
        return f, b0

    def start_gather(c, slot):
        f, _ = chunk_src(c)
        pltpu.async_copy(
            table_hbm.at[f].at[idx_v.at[pl.ds(c * CHUNK, CHUNK)]],
            rows_v.at[slot],
            gsem.at[slot])

    def start_writeback(c, slot):
        f, b0 = chunk_src(c)
        pltpu.async_copy(
            rows_v.at[slot],
            out_hbm.at[f, pl.ds(b0, CHUNK), :],
            wsem.at[slot])

    def wait(sem, slot):
        pltpu.make_async_copy(
            out_hbm.at[0, pl.ds(0, CHUNK), :], rows_v.at[slot], sem.at[slot]
        ).wait()

    start_gather(0, 0)

    def step(c, _):
        slot = lax.rem(c, 2)
        nslot = 1 - slot
        @pl.when(c + 1 < NCH)
        def _():
            @pl.when(c + 1 >= 2)
            def _():
                wait(wsem, nslot)
            start_gather(c + 1, nslot)
        wait(gsem, slot)
        start_writeback(c, slot)
        return 0

    lax.fori_loop(0, NCH, step, 0)
    wait(wsem, 0)
    wait(wsem, 1)


_gather = functools.partial(
    pl.kernel,
    mesh=plsc.VectorSubcoreMesh(core_axis_name="c", subcore_axis_name="s"),
    compiler_params=pltpu.CompilerParams(use_tc_tiling_on_sc=False),
    out_type=jax.ShapeDtypeStruct((N_FIELDS, B, EMBED_DIM), jnp.float32),
    scratch_types=[
        pltpu.VMEM((IPW,), jnp.int32),
        pltpu.VMEM((2, CHUNK, EMBED_DIM), jnp.float32),
        pltpu.SemaphoreType.DMA((2,)),
        pltpu.SemaphoreType.DMA((2,)),
    ],
)(_gather_body)


BB = 512


def _mlp_body(xi_ref, emb_ref, w1d_ref, w1e_ref, b1_ref, w2_ref, b2_ref,
              w3_ref, b3_ref, o_ref):
    x = jnp.concatenate([emb_ref[f] for f in range(N_FIELDS)], axis=-1)
    h1 = jnp.dot(x, w1e_ref[...], preferred_element_type=jnp.float32)
    h1 = h1 + jnp.dot(xi_ref[...], w1d_ref[...], preferred_element_type=jnp.float32)
    h1 = jnp.maximum(h1 + b1_ref[...], 0.0)
    h2 = jnp.dot(h1, w2_ref[...], preferred_element_type=jnp.float32)
    h2 = jnp.maximum(h2 + b2_ref[...], 0.0)
    y = jnp.dot(h2, w3_ref[...], preferred_element_type=jnp.float32) + b3_ref[...]
    o_ref[...] = jax.nn.sigmoid(y)


def kernel(xi, xv, emb_tables, W1, b1, W2, b2, W3, b3):
    xvt = xv.T.reshape(N_FIELDS * B).astype(jnp.int32)

    emb_fm = _gather(xvt, emb_tables)

    W1d = W1[:N_DENSE]
    W1e = W1[N_DENSE:]

    out = pl.pallas_call(
        _mlp_body,
        grid=(B // BB,),
        in_specs=[
            pl.BlockSpec((BB, N_DENSE), lambda i: (i, 0)),
            pl.BlockSpec((N_FIELDS, BB, EMBED_DIM), lambda i: (0, i, 0)),
            pl.BlockSpec((N_DENSE, H1), lambda i: (0, 0)),
            pl.BlockSpec((EIN, H1), lambda i: (0, 0)),
            pl.BlockSpec((1, H1), lambda i: (0, 0)),
            pl.BlockSpec((H1, H2), lambda i: (0, 0)),
            pl.BlockSpec((1, H2), lambda i: (0, 0)),
            pl.BlockSpec((H2, 1), lambda i: (0, 0)),
            pl.BlockSpec((1, 1), lambda i: (0, 0)),
        ],
        out_specs=pl.BlockSpec((BB, 1), lambda i: (i, 0)),
        out_shape=jax.ShapeDtypeStruct((B, 1), jnp.float32),
    )(xi, emb_fm, W1d, W1e, b1.reshape(1, H1), W2, b2.reshape(1, H2),
      W3, b3.reshape(1, 1))
    return out
